# baseline (speedup 1.0000x reference)
import functools

import jax
import jax.numpy as jnp
from jax import lax
from jax.experimental import pallas as pl
from jax.experimental.pallas import tpu as pltpu
from jax.experimental.pallas import tpu_sc as plsc

_NC, _NS = 2, 16
_NW = _NC * _NS
_CHUNK = 512


def kernel(inputs, embedding_encoder):
    batch, hist = inputs.shape
    num_indices = batch * hist
    _, dim = embedding_encoder.shape
    idx = inputs.reshape(num_indices).astype(jnp.int32)
    b_per_w = num_indices // _NW

    mesh = plsc.VectorSubcoreMesh(core_axis_name="c", subcore_axis_name="s")

    @functools.partial(
        pl.kernel,
        mesh=mesh,
        out_type=jax.ShapeDtypeStruct((num_indices, dim), embedding_encoder.dtype),
        scratch_types=[
            pltpu.VMEM_SHARED((_NS, _CHUNK), jnp.int32),
            pltpu.SMEM((_CHUNK,), jnp.int32),
            pltpu.SemaphoreType.DMA,
            pltpu.SemaphoreType.DMA,
        ],
    )
    def gather_kernel(table_hbm, idx_hbm, out_hbm, idx_sh, idx_s, isem, sem):
        sid = lax.axis_index("s")
        wid = sid * _NC + lax.axis_index("c")
        base = wid * b_per_w

        @pl.loop(0, b_per_w, step=_CHUNK)
        def _(off):
            start = base + off
            pltpu.sync_copy(idx_hbm.at[pl.ds(start, _CHUNK)], idx_sh.at[sid])
            pltpu.sync_copy(idx_sh.at[sid], idx_s)

            @pl.loop(0, _CHUNK)
            def _(r):
                j = idx_s[r]
                pltpu.make_async_copy(
                    table_hbm.at[pl.ds(j, 1), :],
                    out_hbm.at[pl.ds(start + r, 1), :],
                    sem,
                ).start()

            @pl.loop(0, _CHUNK)
            def _(r):
                pltpu.make_async_copy(
                    table_hbm.at[pl.ds(0, 1), :],
                    out_hbm.at[pl.ds(start, 1), :],
                    sem,
                ).wait()

    out = gather_kernel(embedding_encoder, idx)
    return out.reshape(batch, hist, dim)


# speedup vs baseline: 7.7469x; 7.7469x over previous
import functools

import jax
import jax.numpy as jnp
from jax import lax
from jax.experimental import pallas as pl
from jax.experimental.pallas import tpu as pltpu
from jax.experimental.pallas import tpu_sc as plsc

_NC, _NS = 2, 16
_NW = _NC * _NS
_WIN = 128


def kernel(inputs, embedding_encoder):
    batch, hist = inputs.shape
    num_indices = batch * hist
    _, dim = embedding_encoder.shape
    idx = inputs.reshape(num_indices).astype(jnp.int32)
    table_pad = jnp.pad(embedding_encoder, ((0, 0), (0, 128 - dim)))
    b_per_w = num_indices // _NW

    mesh = plsc.VectorSubcoreMesh(core_axis_name="c", subcore_axis_name="s")

    @functools.partial(
        pl.kernel,
        mesh=mesh,
        out_type=jax.ShapeDtypeStruct((num_indices, dim), embedding_encoder.dtype),
        scratch_types=[
            pltpu.VMEM((_WIN,), jnp.int32),
            pltpu.VMEM((_WIN, 128), jnp.float32),
            pltpu.VMEM((_WIN, 64), jnp.float32),
            pltpu.SemaphoreType.DMA,
        ],
    )
    def gather_kernel(table_hbm, idx_hbm, out_hbm, idx_v, rows_v, o_v, sem):
        wid = lax.axis_index("s") * _NC + lax.axis_index("c")
        base = wid * b_per_w

        @pl.loop(0, b_per_w, step=_WIN)
        def _(off):
            start = base + off
            pltpu.sync_copy(idx_hbm.at[pl.ds(start, _WIN)], idx_v)
            pltpu.async_copy(table_hbm.at[idx_v], rows_v, sem).wait()

            @pl.loop(0, _WIN)
            def _(r):
                for c in range(0, dim, 16):
                    slc = (pl.ds(r, 1), pl.ds(c, 16))
                    o_v.at[*slc][...] = rows_v.at[*slc][...]

            pltpu.sync_copy(o_v, out_hbm.at[pl.ds(start, _WIN)])

    out = gather_kernel(table_pad, idx)
    return out.reshape(batch, hist, dim)


# speedup vs baseline: 9.1122x; 1.1762x over previous
import functools

import jax
import jax.numpy as jnp
from jax import lax
from jax.experimental import pallas as pl
from jax.experimental.pallas import tpu as pltpu
from jax.experimental.pallas import tpu_sc as plsc

_NC, _NS = 2, 16
_NW = _NC * _NS
_WIN = 128


def kernel(inputs, embedding_encoder):
    batch, hist = inputs.shape
    num_indices = batch * hist
    _, dim = embedding_encoder.shape
    idx = inputs.reshape(num_indices).astype(jnp.int32)
    table_pad = jnp.pad(embedding_encoder, ((0, 0), (0, 128 - dim)))
    b_per_w = num_indices // _NW
    n_win = b_per_w // _WIN

    mesh = plsc.VectorSubcoreMesh(core_axis_name="c", subcore_axis_name="s")

    @functools.partial(
        pl.kernel,
        mesh=mesh,
        out_type=jax.ShapeDtypeStruct((num_indices, dim), embedding_encoder.dtype),
        scratch_types=[
            pltpu.VMEM((_WIN,), jnp.int32),
            pltpu.VMEM((_WIN,), jnp.int32),
            pltpu.VMEM((_WIN, 128), jnp.float32),
            pltpu.VMEM((_WIN, 128), jnp.float32),
            pltpu.VMEM((_WIN, 64), jnp.float32),
            pltpu.VMEM((_WIN, 64), jnp.float32),
            pltpu.SemaphoreType.DMA,
            pltpu.SemaphoreType.DMA,
            pltpu.SemaphoreType.DMA,
            pltpu.SemaphoreType.DMA,
            pltpu.SemaphoreType.DMA,
            pltpu.SemaphoreType.DMA,
        ],
    )
    def gather_kernel(table_hbm, idx_hbm, out_hbm,
                      idx_v0, idx_v1, rows_v0, rows_v1, o_v0, o_v1,
                      isem0, isem1, gsem0, gsem1, osem0, osem1):
        idx_v = (idx_v0, idx_v1)
        rows_v = (rows_v0, rows_v1)
        o_v = (o_v0, o_v1)
        isem = (isem0, isem1)
        gsem = (gsem0, gsem1)
        osem = (osem0, osem1)

        wid = lax.axis_index("s") * _NC + lax.axis_index("c")
        base = wid * b_per_w

        def idx_copy(p, s):
            return pltpu.make_async_copy(
                idx_hbm.at[pl.ds(base + p * _WIN, _WIN)], idx_v[s], isem[s]
            )

        def gather_copy(s):
            return pltpu.make_async_copy(
                table_hbm.at[idx_v[s]], rows_v[s], gsem[s]
            )

        def out_copy(p, s):
            return pltpu.make_async_copy(
                o_v[s], out_hbm.at[pl.ds(base + p * _WIN, _WIN)], osem[s]
            )

        def compact(s):
            @pl.loop(0, _WIN)
            def _(r):
                for c in range(0, dim, 16):
                    o_v[s].at[r, pl.ds(c, 16)][...] = \
                        rows_v[s].at[r, pl.ds(c, 16)][...]

        idx_copy(0, 0).start()
        idx_copy(0, 0).wait()
        gather_copy(0).start()
        idx_copy(1, 1).start()

        @pl.loop(0, n_win, step=2)
        def _(pp):
            for ping in (0, 1):
                p = pp + ping
                s, o = ping, 1 - ping
                gather_copy(s).wait()

                @pl.when(p + 2 < n_win)
                def _():
                    idx_copy(p + 2, s).start()

                @pl.when(p + 1 < n_win)
                def _():
                    idx_copy(p + 1, o).wait()
                    gather_copy(o).start()

                @pl.when(p >= 2)
                def _():
                    out_copy(p - 2, s).wait()

                compact(s)
                out_copy(p, s).start()

        out_copy(n_win - 2, 0).wait()
        out_copy(n_win - 1, 1).wait()

    out = gather_kernel(table_pad, idx)
    return out.reshape(batch, hist, dim)


# speedup vs baseline: 9.8707x; 1.0832x over previous
import functools

import jax
import jax.numpy as jnp
from jax import lax
from jax.experimental import pallas as pl
from jax.experimental.pallas import tpu as pltpu
from jax.experimental.pallas import tpu_sc as plsc

_NC, _NS = 2, 16
_NW = _NC * _NS
_RP = 2
_PSTRIDE = 104


def _prep(inputs, table):
    batch, hist = inputs.shape
    vocab, dim = table.shape

    pad_blk = 20000

    def pad_kernel(tbl_ref, pad_ref):
        blk = tbl_ref[...]
        pad_ref[...] = jnp.concatenate(
            [blk, jnp.zeros((blk.shape[0], 128 - dim), blk.dtype)], axis=1
        )

    table_pad = pl.pallas_call(
        pad_kernel,
        grid=(vocab // pad_blk,),
        in_specs=[pl.BlockSpec((pad_blk, dim), lambda i: (i, 0))],
        out_specs=pl.BlockSpec((pad_blk, 128), lambda i: (i, 0)),
        out_shape=jax.ShapeDtypeStruct((vocab, 128), table.dtype),
    )(table)

    flat_blk = 2048

    def idxpad_kernel(idx_ref, pad_ref):
        blk = idx_ref[...]
        pad_ref[...] = jnp.concatenate(
            [blk, jnp.zeros((blk.shape[0], 128 - hist), blk.dtype)], axis=1
        )

    idx = pl.pallas_call(
        idxpad_kernel,
        grid=(batch // flat_blk,),
        in_specs=[pl.BlockSpec((flat_blk, hist), lambda i: (i, 0))],
        out_specs=pl.BlockSpec((flat_blk, 128), lambda i: (i, 0)),
        out_shape=jax.ShapeDtypeStruct((batch, 128), jnp.int32),
    )(inputs.astype(jnp.int32))

    return table_pad, idx


def kernel(inputs, embedding_encoder):
    batch, hist = inputs.shape
    num_indices = batch * hist
    _, dim = embedding_encoder.shape
    table_pad, idx = _prep(inputs, embedding_encoder)
    rows_per_w = batch // _NW
    n_win = rows_per_w // _RP
    win = _RP * hist

    mesh = plsc.VectorSubcoreMesh(core_axis_name="c", subcore_axis_name="s")

    @functools.partial(
        pl.kernel,
        mesh=mesh,
        out_type=jax.ShapeDtypeStruct((batch, hist, dim), embedding_encoder.dtype),
        scratch_types=[
            pltpu.VMEM((_RP, 128), jnp.int32),
            pltpu.VMEM((_RP, 128), jnp.int32),
            pltpu.VMEM((win, 128), jnp.float32),
            pltpu.VMEM((win, 128), jnp.float32),
            pltpu.VMEM((_RP, hist, dim), jnp.float32),
            pltpu.VMEM((_RP, hist, dim), jnp.float32),
            pltpu.SemaphoreType.DMA,
            pltpu.SemaphoreType.DMA,
            pltpu.SemaphoreType.DMA,
            pltpu.SemaphoreType.DMA,
            pltpu.SemaphoreType.DMA,
            pltpu.SemaphoreType.DMA,
        ],
    )
    def gather_kernel(table_hbm, idx_hbm, out_hbm,
                      idx_v0, idx_v1, rows_v0, rows_v1, o_v0, o_v1,
                      isem0, isem1, gsem0, gsem1, osem0, osem1):
        idx_v = (idx_v0, idx_v1)
        rows_v = (rows_v0, rows_v1)
        o_v = (o_v0, o_v1)
        isem = (isem0, isem1)
        gsem = (gsem0, gsem1)
        osem = (osem0, osem1)

        wid = lax.axis_index("s") * _NC + lax.axis_index("c")
        base_row = wid * rows_per_w

        def idx_copies(p, s):
            row = base_row + p * _RP
            return [
                pltpu.make_async_copy(
                    idx_hbm.at[pl.ds(row + r, 1), :],
                    idx_v[s].at[pl.ds(r, 1), :],
                    isem[s],
                )
                for r in range(_RP)
            ]

        def gather_copies(s):
            return [
                pltpu.make_async_copy(
                    table_hbm.at[idx_v[s].at[r, pl.ds(0, hist)]],
                    rows_v[s].at[pl.ds(r * hist, hist)],
                    gsem[s],
                )
                for r in range(_RP)
            ]

        def out_copy(p, s):
            return pltpu.make_async_copy(
                o_v[s], out_hbm.at[pl.ds(base_row + p * _RP, _RP)], osem[s]
            )

        def compact(s):
            for r0 in range(_RP):
                @pl.loop(0, hist)
                def _(r):
                    for c in range(0, dim, 16):
                        o_v[s].at[r0, r, pl.ds(c, 16)][...] = \
                            rows_v[s].at[r0 * hist + r, pl.ds(c, 16)][...]

        for cp in idx_copies(0, 0):
            cp.start()
        for cp in idx_copies(0, 0):
            cp.wait()
        for cp in gather_copies(0):
            cp.start()
        for cp in idx_copies(1, 1):
            cp.start()

        @pl.loop(0, n_win, step=2)
        def _(pp):
            for ping in (0, 1):
                p = pp + ping
                s, o = ping, 1 - ping
                for cp in gather_copies(s):
                    cp.wait()

                @pl.when(p + 2 < n_win)
                def _():
                    for cp in idx_copies(p + 2, s):
                        cp.start()

                @pl.when(p + 1 < n_win)
                def _():
                    for cp in idx_copies(p + 1, o):
                        cp.wait()
                    for cp in gather_copies(o):
                        cp.start()

                @pl.when(p >= 2)
                def _():
                    out_copy(p - 2, s).wait()

                compact(s)
                out_copy(p, s).start()

        out_copy(n_win - 2, 0).wait()
        out_copy(n_win - 1, 1).wait()

    return gather_kernel(table_pad, idx)


# speedup vs baseline: 10.9290x; 1.1072x over previous
import functools

import jax
import jax.numpy as jnp
from jax import lax
from jax.experimental import pallas as pl
from jax.experimental.pallas import tpu as pltpu
from jax.experimental.pallas import tpu_sc as plsc

_NC, _NS = 2, 16
_NW = _NC * _NS
_RP = 2
_PSTRIDE = 104


def _prep(inputs, table):
    batch, hist = inputs.shape
    vocab, dim = table.shape

    table_pad = jnp.pad(table, ((0, 0), (0, 128 - dim)))

    flat_blk = 2048

    def idxpad_kernel(idx_ref, pad_ref):
        blk = idx_ref[...]
        pad_ref[...] = jnp.concatenate(
            [blk, jnp.zeros((blk.shape[0], 128 - hist), blk.dtype)], axis=1
        )

    idx = pl.pallas_call(
        idxpad_kernel,
        grid=(batch // flat_blk,),
        in_specs=[pl.BlockSpec((flat_blk, hist), lambda i: (i, 0))],
        out_specs=pl.BlockSpec((flat_blk, 128), lambda i: (i, 0)),
        out_shape=jax.ShapeDtypeStruct((batch, 128), jnp.int32),
    )(inputs.astype(jnp.int32))

    return table_pad, idx


def kernel(inputs, embedding_encoder):
    batch, hist = inputs.shape
    num_indices = batch * hist
    _, dim = embedding_encoder.shape
    table_pad, idx = _prep(inputs, embedding_encoder)
    rows_per_w = batch // _NW
    n_win = rows_per_w // _RP
    win = _RP * hist

    mesh = plsc.VectorSubcoreMesh(core_axis_name="c", subcore_axis_name="s")

    @functools.partial(
        pl.kernel,
        mesh=mesh,
        out_type=jax.ShapeDtypeStruct((batch, hist, dim), embedding_encoder.dtype),
        scratch_types=[
            pltpu.VMEM((_RP, 128), jnp.int32),
            pltpu.VMEM((_RP, 128), jnp.int32),
            pltpu.VMEM((win, 128), jnp.float32),
            pltpu.VMEM((win, 128), jnp.float32),
            pltpu.VMEM((_RP, hist, dim), jnp.float32),
            pltpu.VMEM((_RP, hist, dim), jnp.float32),
            pltpu.SemaphoreType.DMA,
            pltpu.SemaphoreType.DMA,
            pltpu.SemaphoreType.DMA,
            pltpu.SemaphoreType.DMA,
            pltpu.SemaphoreType.DMA,
            pltpu.SemaphoreType.DMA,
        ],
    )
    def gather_kernel(table_hbm, idx_hbm, out_hbm,
                      idx_v0, idx_v1, rows_v0, rows_v1, o_v0, o_v1,
                      isem0, isem1, gsem0, gsem1, osem0, osem1):
        idx_v = (idx_v0, idx_v1)
        rows_v = (rows_v0, rows_v1)
        o_v = (o_v0, o_v1)
        isem = (isem0, isem1)
        gsem = (gsem0, gsem1)
        osem = (osem0, osem1)

        wid = lax.axis_index("s") * _NC + lax.axis_index("c")
        base_row = wid * rows_per_w

        def idx_copies(p, s):
            row = base_row + p * _RP
            return [
                pltpu.make_async_copy(
                    idx_hbm.at[pl.ds(row + r, 1), :],
                    idx_v[s].at[pl.ds(r, 1), :],
                    isem[s],
                )
                for r in range(_RP)
            ]

        def gather_copies(s):
            return [
                pltpu.make_async_copy(
                    table_hbm.at[idx_v[s].at[r, pl.ds(0, hist)]],
                    rows_v[s].at[pl.ds(r * hist, hist)],
                    gsem[s],
                )
                for r in range(_RP)
            ]

        def out_copy(p, s):
            return pltpu.make_async_copy(
                o_v[s], out_hbm.at[pl.ds(base_row + p * _RP, _RP)], osem[s]
            )

        def compact(s):
            for r0 in range(_RP):
                @pl.loop(0, hist)
                def _(r):
                    for c in range(0, dim, 16):
                        o_v[s].at[r0, r, pl.ds(c, 16)][...] = \
                            rows_v[s].at[r0 * hist + r, pl.ds(c, 16)][...]

        for cp in idx_copies(0, 0):
            cp.start()
        for cp in idx_copies(0, 0):
            cp.wait()
        for cp in gather_copies(0):
            cp.start()
        for cp in idx_copies(1, 1):
            cp.start()

        @pl.loop(0, n_win, step=2)
        def _(pp):
            for ping in (0, 1):
                p = pp + ping
                s, o = ping, 1 - ping
                for cp in gather_copies(s):
                    cp.wait()

                @pl.when(p + 2 < n_win)
                def _():
                    for cp in idx_copies(p + 2, s):
                        cp.start()

                @pl.when(p + 1 < n_win)
                def _():
                    for cp in idx_copies(p + 1, o):
                        cp.wait()
                    for cp in gather_copies(o):
                        cp.start()

                @pl.when(p >= 2)
                def _():
                    out_copy(p - 2, s).wait()

                compact(s)
                out_copy(p, s).start()

        out_copy(n_win - 2, 0).wait()
        out_copy(n_win - 1, 1).wait()

    return gather_kernel(table_pad, idx)
